# baseline (speedup 1.0000x reference)
import functools

import jax
import jax.numpy as jnp
from jax import lax
from jax.experimental import pallas as pl
from jax.experimental.pallas import tpu as pltpu
from jax.experimental.pallas import tpu_sc as plsc

V = 1000000
D = 64
B = 16384
NN = 5
NW = 32
BPW = B // NW
NCH = BPW // 128

RCOLS = 2048
RGRID = -(-V // RCOLS)


def _repack_body(x_ref, y_ref):
    x = x_ref[...]
    y_ref[...] = x.reshape(D, RCOLS // 2, 2).transpose(1, 2, 0).reshape(
        RCOLS // 2, 2 * D)


_repack = pl.pallas_call(
    _repack_body,
    grid=(RGRID,),
    in_specs=[pl.BlockSpec((D, RCOLS), lambda g: (0, g))],
    out_specs=pl.BlockSpec((RCOLS // 2, 2 * D), lambda g: (g, 0)),
    out_shape=jax.ShapeDtypeStruct((V // 2, 2 * D), jnp.float32),
)


def _sc_kernel():
    mesh = plsc.VectorSubcoreMesh(core_axis_name="c", subcore_axis_name="s")

    @functools.partial(
        pl.kernel,
        mesh=mesh,
        compiler_params=pltpu.CompilerParams(needs_layout_passes=False),
        out_type=jax.ShapeDtypeStruct((NN, B // 128, 128), jnp.float32),
        scratch_types=[
            pltpu.VMEM((NCH, 128), jnp.int32),
            pltpu.VMEM((NCH, 128), jnp.int32),
            pltpu.VMEM((NCH, 128), jnp.int32),
            pltpu.VMEM((BPW, 128), jnp.float32),
            pltpu.VMEM((128, 128), jnp.float32),
            pltpu.VMEM((NCH, 128), jnp.float32),
            pltpu.SemaphoreType.DMA,
        ],
    )
    def k(tgt_hbm, ctx_hbm, wt_hbm, wc_hbm, out_hbm, idx_t, idx_c, idx_h,
          rows_t, rows_c, dots_v, sem):
        wid = lax.axis_index("s") * 2 + lax.axis_index("c")
        crow = wid * NCH
        lanes = lax.iota(jnp.int32, 16)

        def halve(src, dst):
            for r in range(NCH):
                for c in range(128 // 16):
                    dst[r, pl.ds(c * 16, 16)] = (
                        src[r, pl.ds(c * 16, 16)] >> 1)

        pltpu.sync_copy(tgt_hbm.at[pl.ds(crow, NCH)], idx_t)
        halve(idx_t, idx_h)
        for j in range(NCH):
            pltpu.async_copy(
                wt_hbm.at[idx_h.at[j]],
                rows_t.at[pl.ds(j * 128, 128)], sem)
        for j in range(NCH):
            pltpu.make_async_copy(
                wt_hbm.at[idx_h.at[0]],
                rows_t.at[pl.ds(0, 128)], sem).wait()

        def dot_group(j, g, _):
            res = jnp.zeros((16,), jnp.float32)
            ht_vec = (idx_t[j, pl.ds(g * 16, 16)] & 1) * 64
            hc_vec = (idx_c[j, pl.ds(g * 16, 16)] & 1) * 64
            for i in range(16):
                p = g * 16 + i
                b = j * 128 + p
                ht = ht_vec[i]
                hc = hc_vec[i]
                acc = None
                for dc in range(D // 16):
                    we = rows_t[b, pl.ds(ht + dc * 16, 16)]
                    ce = rows_c[p, pl.ds(hc + dc * 16, 16)]
                    acc = we * ce if acc is None else acc + we * ce
                res = jnp.where(lanes == i, jnp.sum(acc), res)
            dots_v[j, pl.ds(g * 16, 16)] = res
            return _

        def chunk_body(j, _):
            pltpu.async_copy(wc_hbm.at[idx_h.at[j]], rows_c, sem)
            pltpu.make_async_copy(
                wc_hbm.at[idx_h.at[0]], rows_c, sem).wait()
            lax.fori_loop(0, 128 // 16,
                          lambda g, c: dot_group(j, g, c), 0)
            return _

        for n in range(NN):
            pltpu.sync_copy(ctx_hbm.at[n, pl.ds(crow, NCH)], idx_c)
            halve(idx_c, idx_h)
            lax.fori_loop(0, NCH, chunk_body, 0)
            pltpu.sync_copy(dots_v, out_hbm.at[n, pl.ds(crow, NCH)])

    return k


_k = _sc_kernel()


def kernel(target, context, W_target, W_context):
    tgt2 = target.reshape(B // 128, 128)
    ctx3 = context.reshape(B, NN).T.reshape(NN, B // 128, 128)
    wt2 = _repack(W_target.T)
    wc2 = _repack(W_context.T)
    out = _k(tgt2, ctx3, wt2, wc2)
    return out.reshape(NN, B).T


# speedup vs baseline: 18.2329x; 18.2329x over previous
import functools

import jax
import jax.numpy as jnp
from jax import lax
from jax.experimental import pallas as pl
from jax.experimental.pallas import tpu as pltpu
from jax.experimental.pallas import tpu_sc as plsc

V = 1000000
D = 64
B = 16384
NN = 5
NW = 32
BPW = B // NW
NCH = BPW // 128

RCOLS = 2048
RHALF = RCOLS // 2
RGRID = -(-V // RCOLS)
VPACK = RGRID * RHALF


def _repack_body(x_ref, y_ref):
    xt = x_ref[...].T
    y_ref[:, 0:D] = xt[0:RHALF]
    y_ref[:, D:2 * D] = xt[RHALF:RCOLS]


_repack = pl.pallas_call(
    _repack_body,
    grid=(RGRID,),
    in_specs=[pl.BlockSpec((D, RCOLS), lambda g: (0, g))],
    out_specs=pl.BlockSpec((RHALF, 2 * D), lambda g: (g, 0)),
    out_shape=jax.ShapeDtypeStruct((VPACK, 2 * D), jnp.float32),
)


def _sc_kernel():
    mesh = plsc.VectorSubcoreMesh(core_axis_name="c", subcore_axis_name="s")

    @functools.partial(
        pl.kernel,
        mesh=mesh,
        compiler_params=pltpu.CompilerParams(needs_layout_passes=False),
        out_type=jax.ShapeDtypeStruct((NN, B // 128, 128), jnp.float32),
        scratch_types=[
            pltpu.VMEM((NCH, 128), jnp.int32),
            pltpu.VMEM((NCH, 128), jnp.int32),
            pltpu.VMEM((NCH, 128), jnp.int32),
            pltpu.VMEM((BPW, 128), jnp.float32),
            pltpu.VMEM((128, 128), jnp.float32),
            pltpu.VMEM((NCH, 128), jnp.float32),
            pltpu.SemaphoreType.DMA,
        ],
    )
    def k(tgt_hbm, ctx_hbm, wt_hbm, wc_hbm, out_hbm, idx_t, idx_c, idx_h,
          rows_t, rows_c, dots_v, sem):
        wid = lax.axis_index("s") * 2 + lax.axis_index("c")
        crow = wid * NCH
        lanes = lax.iota(jnp.int32, 16)

        def halve(src, dst):
            for r in range(NCH):
                for c in range(128 // 16):
                    v = src[r, pl.ds(c * 16, 16)]
                    dst[r, pl.ds(c * 16, 16)] = (
                        ((v >> 11) << 10) + (v & 1023))

        pltpu.sync_copy(tgt_hbm.at[pl.ds(crow, NCH)], idx_t)
        halve(idx_t, idx_h)
        for j in range(NCH):
            pltpu.async_copy(
                wt_hbm.at[idx_h.at[j]],
                rows_t.at[pl.ds(j * 128, 128)], sem)
        for j in range(NCH):
            pltpu.make_async_copy(
                wt_hbm.at[idx_h.at[0]],
                rows_t.at[pl.ds(0, 128)], sem).wait()

        def dot_group(j, g, _):
            res = jnp.zeros((16,), jnp.float32)
            ht_vec = ((idx_t[j, pl.ds(g * 16, 16)] >> 10) & 1) * 64
            hc_vec = ((idx_c[j, pl.ds(g * 16, 16)] >> 10) & 1) * 64
            for i in range(16):
                p = g * 16 + i
                b = j * 128 + p
                ht = ht_vec[i]
                hc = hc_vec[i]
                acc = None
                for dc in range(D // 16):
                    we = rows_t[b, pl.ds(ht + dc * 16, 16)]
                    ce = rows_c[p, pl.ds(hc + dc * 16, 16)]
                    acc = we * ce if acc is None else acc + we * ce
                res = jnp.where(lanes == i, jnp.sum(acc), res)
            dots_v[j, pl.ds(g * 16, 16)] = res
            return _

        def chunk_body(j, _):
            pltpu.async_copy(wc_hbm.at[idx_h.at[j]], rows_c, sem)
            pltpu.make_async_copy(
                wc_hbm.at[idx_h.at[0]], rows_c, sem).wait()
            lax.fori_loop(0, 128 // 16,
                          lambda g, c: dot_group(j, g, c), 0)
            return _

        for n in range(NN):
            pltpu.sync_copy(ctx_hbm.at[n, pl.ds(crow, NCH)], idx_c)
            halve(idx_c, idx_h)
            lax.fori_loop(0, NCH, chunk_body, 0)
            pltpu.sync_copy(dots_v, out_hbm.at[n, pl.ds(crow, NCH)])

    return k


_k = _sc_kernel()


def kernel(target, context, W_target, W_context):
    tgt2 = target.reshape(B // 128, 128)
    ctx3 = context.reshape(B, NN).T.reshape(NN, B // 128, 128)
    wt2 = _repack(W_target.T)
    wc2 = _repack(W_context.T)
    out = _k(tgt2, ctx3, wt2, wc2)
    return out.reshape(NN, B).T
